# Initial kernel scaffold; baseline (speedup 1.0000x reference)
import jax
import jax.numpy as jnp
from jax.experimental import pallas as pl


def kernel(features_rank, train_features, train_labels):
    raise NotImplementedError("write your pallas kernel here")


# speedup vs baseline: 1.2324x; 1.2324x over previous
import functools

import jax
import jax.numpy as jnp
from jax.experimental import pallas as pl
from jax.experimental.pallas import tpu as pltpu

NB_K = (10, 20, 100)
MAXK = 100
TEMP = 0.07
NCLS = 1000
NCLS_PAD = 1024
BQ = 128
NC = 128
NEG = -3.0e38


def _cx_level(v, lab, li, j, k):
    n = v.shape[1]
    bitj = (li & j) != 0
    pv = jnp.where(bitj, pltpu.roll(v, j, 1), pltpu.roll(v, n - j, 1))
    plb = jnp.where(bitj, pltpu.roll(lab, j, 1), pltpu.roll(lab, n - j, 1))
    i_lt_p = (li & j) == 0
    if k is None:
        take_min = jnp.logical_not(i_lt_p)
    else:
        up = (li & k) == 0
        take_min = jnp.logical_not(jnp.logical_xor(i_lt_p, up))
    take_max = jnp.logical_not(take_min)
    chose = (take_min & (pv < v)) | (take_max & (pv > v))
    return jnp.where(chose, pv, v), jnp.where(chose, plb, lab)


def _sort_asc(v, lab, li):
    n = v.shape[1]
    k = 2
    while k <= n:
        j = k // 2
        while j >= 1:
            v, lab = _cx_level(v, lab, li, j, k)
            j //= 2
        k *= 2
    return v, lab


def _merge_desc(v, lab, li):
    j = v.shape[1] // 2
    while j >= 1:
        v, lab = _cx_level(v, lab, li, j, None)
        j //= 2
    return v, lab


def _body(n_train, nb, q_ref, tr_ref, lab_ref, out_ref, rv_ref, rl_ref):
    ni = pl.program_id(1)

    @pl.when(ni == 0)
    def _init():
        rv_ref[...] = jnp.full((BQ, NC), NEG, dtype=jnp.float32)
        rl_ref[...] = jnp.zeros((BQ, NC), dtype=jnp.int32)

    li = jax.lax.broadcasted_iota(jnp.int32, (1, NC), 1)

    sim = jax.lax.dot_general(
        q_ref[...], tr_ref[...],
        dimension_numbers=(((1,), (1,)), ((), ())),
        preferred_element_type=jnp.float32,
    )
    col = ni * NC + li
    sim = jnp.where(col < n_train, sim, NEG)
    lab = jnp.broadcast_to(lab_ref[0, :, :], (BQ, NC)).astype(jnp.int32)

    sim, lab = _sort_asc(sim, lab, li)

    rv = rv_ref[...]
    rl = rl_ref[...]
    chose = sim > rv
    mv = jnp.where(chose, sim, rv)
    ml = jnp.where(chose, lab, rl)
    mv, ml = _merge_desc(mv, ml, li)
    rv_ref[...] = mv
    rl_ref[...] = ml

    @pl.when(ni == nb - 1)
    def _vote():
        v = rv_ref[...]
        lb = rl_ref[...]
        m = v[:, 0:1]
        lanemask = li < MAXK
        e = jnp.where(lanemask, jnp.exp((v - m) / TEMP), 0.0)
        z = jnp.sum(e, axis=1, keepdims=True)
        w = e / z
        ci = jax.lax.broadcasted_iota(jnp.int32, (BQ, NCLS_PAD), 1)

        def step(_, carry):
            acc, wc, lc = carry
            wj = wc[:, 0:1]
            lj = lc[:, 0:1]
            acc = acc + wj * (ci == lj).astype(jnp.float32)
            return acc, pltpu.roll(wc, NC - 1, 1), pltpu.roll(lc, NC - 1, 1)

        acc = jnp.zeros((BQ, NCLS_PAD), dtype=jnp.float32)
        acc, w, lb = jax.lax.fori_loop(0, NB_K[0], step, (acc, w, lb))
        out_ref[0, :, :] = acc[:, :NCLS]
        acc, w, lb = jax.lax.fori_loop(NB_K[0], NB_K[1], step, (acc, w, lb))
        out_ref[1, :, :] = acc[:, :NCLS]
        acc, w, lb = jax.lax.fori_loop(NB_K[1], NB_K[2], step, (acc, w, lb))
        out_ref[2, :, :] = acc[:, :NCLS]


def kernel(features_rank, train_features, train_labels):
    nq, d = features_rank.shape
    n_train = train_features.shape[0]
    nb = (n_train + NC - 1) // NC
    n_pad = nb * NC

    labs = jnp.pad(train_labels, (0, n_pad - n_train)).reshape(nb, 1, NC)

    grid = (nq // BQ, nb)
    out = pl.pallas_call(
        functools.partial(_body, n_train, nb),
        grid=grid,
        in_specs=[
            pl.BlockSpec((BQ, d), lambda qi, ni: (qi, 0)),
            pl.BlockSpec((NC, d), lambda qi, ni: (ni, 0)),
            pl.BlockSpec((1, 1, NC), lambda qi, ni: (ni, 0, 0)),
        ],
        out_specs=pl.BlockSpec((3, BQ, NCLS), lambda qi, ni: (0, qi, 0)),
        out_shape=jax.ShapeDtypeStruct((3, nq, NCLS), jnp.float32),
        scratch_shapes=[
            pltpu.VMEM((BQ, NC), jnp.float32),
            pltpu.VMEM((BQ, NC), jnp.int32),
        ],
        compiler_params=pltpu.CompilerParams(
            dimension_semantics=("parallel", "arbitrary"),
        ),
    )(features_rank, train_features, labs)
    return out
